# baseline (speedup 1.0000x reference)
import jax
import jax.numpy as jnp
from jax import lax
from jax.experimental import pallas as pl
from jax.experimental.pallas import tpu as pltpu
from jax.experimental.pallas import tpu_sc as plsc

D = 1024
EPS = 1e-6
NW = 32
NTOK = 16384
K = 1
CH = NTOK // K
ROWS_PER_W = CH // NW
C = 16
G = ROWS_PER_W // C
NBUF = 4
BR = 2048


def _gather_body(table_h, idx_h, out_h, idx_v, rows_v, gsems, ssems):
    cid = lax.axis_index("c")
    sid = lax.axis_index("s")
    wid = sid * 2 + cid
    base = wid * ROWS_PER_W

    pltpu.sync_copy(idx_h.at[pl.ds(base, ROWS_PER_W)], idx_v)

    def gather_copy(g, b):
        return pltpu.make_async_copy(
            table_h.at[idx_v.at[pl.ds(g * C, C)]], rows_v.at[b], gsems.at[b]
        )

    def store_copy(g, b):
        return pltpu.make_async_copy(
            rows_v.at[b], out_h.at[pl.ds(base + g * C, C)], ssems.at[b]
        )

    gather_copy(0, 0).start()
    gather_copy(1, 1).start()
    for g in range(G):
        b = g % NBUF
        if g >= 2:
            store_copy(g - 2, (g - 2) % NBUF).wait()
        if g + 2 < G:
            gather_copy(g + 2, (g + 2) % NBUF).start()
        gather_copy(g, b).wait()
        store_copy(g, b).start()
    for g in range(max(G - 2, 0), G):
        store_copy(g, g % NBUF).wait()


def _sc_gather(table, idx_chunk):
    mesh = plsc.VectorSubcoreMesh(core_axis_name="c", subcore_axis_name="s")
    return pl.kernel(
        _gather_body,
        out_type=jax.ShapeDtypeStruct((CH, D), jnp.float32),
        mesh=mesh,
        scratch_types=[
            pltpu.VMEM((ROWS_PER_W,), jnp.int32),
            pltpu.VMEM((NBUF, C, D), jnp.float32),
            pltpu.SemaphoreType.DMA((NBUF,)),
            pltpu.SemaphoreType.DMA((NBUF,)),
        ],
    )(table, idx_chunk)


def _ln_body(x_ref, g_ref, b_ref, o_ref):
    x = x_ref[...]
    m = jnp.mean(x, axis=-1, keepdims=True)
    m2 = jnp.mean(x * x, axis=-1, keepdims=True)
    r = lax.rsqrt(m2 - m * m + EPS)
    o_ref[...] = (x - m) * (r * g_ref[...]) + b_ref[...]


def _tc_ln(x, gamma, beta):
    return pl.pallas_call(
        _ln_body,
        grid=(CH // BR,),
        in_specs=[
            pl.BlockSpec((BR, D), lambda i: (i, 0)),
            pl.BlockSpec((D,), lambda i: (0,)),
            pl.BlockSpec((D,), lambda i: (0,)),
        ],
        out_specs=pl.BlockSpec((BR, D), lambda i: (i, 0)),
        out_shape=jax.ShapeDtypeStruct((CH, D), jnp.float32),
    )(x, gamma, beta)


@jax.jit
def _emb_ln(table, idx, gamma, beta):
    idx_chunks = idx.reshape(K, CH)
    gathered = [_sc_gather(table, idx_chunks[k]) for k in range(K)]
    outs = [_tc_ln(g, gamma, beta) for g in gathered]
    return jnp.concatenate(outs, axis=0)


def kernel(src, table, gamma, beta):
    idx = src.reshape(-1).astype(jnp.int32)
    out = _emb_ln(table, idx, gamma, beta)
    return out.reshape(src.shape + (D,))


# speedup vs baseline: 1.0370x; 1.0370x over previous
import jax
import jax.numpy as jnp
from jax import lax
from jax.experimental import pallas as pl
from jax.experimental.pallas import tpu as pltpu
from jax.experimental.pallas import tpu_sc as plsc

D = 1024
EPS = 1e-6
NW = 32
NTOK = 16384
K = 1
CH = NTOK // K
ROWS_PER_W = CH // NW
C = 16
G = ROWS_PER_W // C
NBUF = 6
BR = 2048


def _gather_body(table_h, idx_h, out_h, idx_v, rows_v, gsems, ssems):
    cid = lax.axis_index("c")
    sid = lax.axis_index("s")
    wid = sid * 2 + cid
    base = wid * ROWS_PER_W

    pltpu.sync_copy(idx_h.at[pl.ds(base, ROWS_PER_W)], idx_v)

    def gather_copy(g, b):
        return pltpu.make_async_copy(
            table_h.at[idx_v.at[pl.ds(g * C, C)]], rows_v.at[b], gsems.at[b]
        )

    def store_copy(g, b):
        return pltpu.make_async_copy(
            rows_v.at[b], out_h.at[pl.ds(base + g * C, C)], ssems.at[b]
        )

    DIST = 3
    for d in range(DIST):
        gather_copy(d, d % NBUF).start()
    for g in range(G):
        b = g % NBUF
        if g >= NBUF - DIST:
            store_copy(g - (NBUF - DIST), (g - (NBUF - DIST)) % NBUF).wait()
        if g + DIST < G:
            gather_copy(g + DIST, (g + DIST) % NBUF).start()
        gather_copy(g, b).wait()
        store_copy(g, b).start()
    for g in range(max(G - (NBUF - DIST), 0), G):
        store_copy(g, g % NBUF).wait()


def _sc_gather(table, idx_chunk):
    mesh = plsc.VectorSubcoreMesh(core_axis_name="c", subcore_axis_name="s")
    return pl.kernel(
        _gather_body,
        out_type=jax.ShapeDtypeStruct((CH, D), jnp.float32),
        mesh=mesh,
        scratch_types=[
            pltpu.VMEM((ROWS_PER_W,), jnp.int32),
            pltpu.VMEM((NBUF, C, D), jnp.float32),
            pltpu.SemaphoreType.DMA((NBUF,)),
            pltpu.SemaphoreType.DMA((NBUF,)),
        ],
    )(table, idx_chunk)


def _ln_body(x_ref, g_ref, b_ref, o_ref):
    x = x_ref[...]
    m = jnp.mean(x, axis=-1, keepdims=True)
    xc = x - m
    var = jnp.mean(xc * xc, axis=-1, keepdims=True)
    o_ref[...] = xc * lax.rsqrt(var + EPS) * g_ref[...] + b_ref[...]


def _tc_ln(x, gamma, beta):
    return pl.pallas_call(
        _ln_body,
        grid=(CH // BR,),
        in_specs=[
            pl.BlockSpec((BR, D), lambda i: (i, 0)),
            pl.BlockSpec((D,), lambda i: (0,)),
            pl.BlockSpec((D,), lambda i: (0,)),
        ],
        out_specs=pl.BlockSpec((BR, D), lambda i: (i, 0)),
        out_shape=jax.ShapeDtypeStruct((CH, D), jnp.float32),
    )(x, gamma, beta)


@jax.jit
def _emb_ln(table, idx, gamma, beta):
    idx_chunks = idx.reshape(K, CH)
    gathered = [_sc_gather(table, idx_chunks[k]) for k in range(K)]
    outs = [_tc_ln(g, gamma, beta) for g in gathered]
    return jnp.concatenate(outs, axis=0)


def kernel(src, table, gamma, beta):
    idx = src.reshape(-1).astype(jnp.int32)
    out = _emb_ln(table, idx, gamma, beta)
    return out.reshape(src.shape + (D,))
